# baseline (speedup 1.0000x reference)
import functools

import jax
import jax.numpy as jnp
from jax import lax
from jax.experimental import pallas as pl
from jax.experimental.pallas import tpu as pltpu
from jax.experimental.pallas import tpu_sc as plsc

_BATCH = 16384
_DIM = 64
_NREL = 1000
_L = 16
_NC = 2
_NS = 16
_NW = _NC * _NS
_BPW = _BATCH // _NW
_G = 16
_NGRP = _BPW // _G

_mesh = plsc.VectorSubcoreMesh(core_axis_name="c", subcore_axis_name="s")


def _gather_body(h_hbm, t_hbm, ent_hbm, g_hbm, hv, tv, sem):
    wid = lax.axis_index("s") * _NC + lax.axis_index("c")
    base = wid * _BPW
    pltpu.sync_copy(h_hbm.at[wid], hv)
    pltpu.sync_copy(t_hbm.at[wid], tv)

    def body(m, carry):
        gsl = pl.ds(m * _G, _G)
        hvv = hv[gsl] >> 3
        tvv = tv[gsl] >> 3
        for n in range(_G):
            pos = base + m * _G + n
            pltpu.async_copy(ent_hbm.at[hvv[n]], g_hbm.at[pos], sem)
            pltpu.async_copy(ent_hbm.at[tvv[n]], g_hbm.at[_BATCH + pos],
                             sem)
        for n in range(_G):
            pltpu.make_async_copy(ent_hbm.at[0], g_hbm.at[base],
                                  sem).wait()
            pltpu.make_async_copy(ent_hbm.at[0], g_hbm.at[base],
                                  sem).wait()
        return carry

    lax.fori_loop(0, _NGRP, body, 0)


_gather = functools.partial(
    pl.kernel,
    mesh=_mesh,
    out_type=jax.ShapeDtypeStruct((2 * _BATCH, 8, _DIM), jnp.float32),
    scratch_types=[
        pltpu.VMEM((_BPW,), jnp.int32),
        pltpu.VMEM((_BPW,), jnp.int32),
        pltpu.SemaphoreType.DMA,
    ],
    compiler_params=pltpu.CompilerParams(use_tc_tiling_on_sc=True),
)(_gather_body)


def _combine_body(h_hbm, r_hbm, t_hbm, g2_hbm, rel_hbm, out_hbm,
                  hv, rv, tv, hbuf, tbuf, relv, ov, sem_a, sem_b):
    wid = lax.axis_index("s") * _NC + lax.axis_index("c")
    base = wid * _BPW
    pltpu.sync_copy(h_hbm.at[pl.ds(base, _BPW)], hv)
    pltpu.sync_copy(r_hbm.at[pl.ds(base, _BPW)], rv)
    pltpu.sync_copy(t_hbm.at[pl.ds(base, _BPW)], tv)
    pltpu.sync_copy(rel_hbm, relv)

    def fire(g, slot, sem):
        dst = pl.ds(slot * _G, _G)
        pltpu.async_copy(g2_hbm.at[pl.ds(base + g * _G, _G)],
                         hbuf.at[dst], sem)
        pltpu.async_copy(g2_hbm.at[pl.ds(_BATCH + base + g * _G, _G)],
                         tbuf.at[dst], sem)

    def drain(slot, sem):
        dst = pl.ds(slot * _G, _G)
        pltpu.make_async_copy(g2_hbm.at[pl.ds(0, _G)], hbuf.at[dst],
                              sem).wait()
        pltpu.make_async_copy(g2_hbm.at[pl.ds(0, _G)], tbuf.at[dst],
                              sem).wait()

    def compute(g, slot):
        gsl = pl.ds(g * _G, _G)
        hvv = hv[gsl]
        rvv = rv[gsl]
        tvv = tv[gsl]
        for n in range(_G):
            hs = (hvv[n] & 7) << 6
            ts = (tvv[n] & 7) << 6
            rw = rvv[n]
            rg = rw >> 3
            rs = (rw & 7) << 6
            row = slot * _G + n
            for c in range(_DIM // _L):
                sl = c * _L
                he = hbuf[row, pl.ds(pl.multiple_of(hs + sl, _L), _L)]
                te = tbuf[row, pl.ds(pl.multiple_of(ts + sl, _L), _L)]
                re = relv[rg, pl.ds(pl.multiple_of(rs + sl, _L), _L)]
                ov[n, pl.ds(sl, _L)] = he + re - te
        pltpu.sync_copy(ov, out_hbm.at[pl.ds(base + g * _G, _G)])

    fire(0, 0, sem_a)

    def body(m, carry):
        g0 = 2 * m
        fire(g0 + 1, 1, sem_b)
        drain(0, sem_a)
        compute(g0, 0)

        @pl.when(m < _NGRP // 2 - 1)
        def _():
            fire(g0 + 2, 0, sem_a)

        drain(1, sem_b)
        compute(g0 + 1, 1)
        return carry

    lax.fori_loop(0, _NGRP // 2, body, 0)


_combine = functools.partial(
    pl.kernel,
    mesh=_mesh,
    out_type=jax.ShapeDtypeStruct((_BATCH, _DIM), jnp.float32),
    scratch_types=[
        pltpu.VMEM((_BPW,), jnp.int32),
        pltpu.VMEM((_BPW,), jnp.int32),
        pltpu.VMEM((_BPW,), jnp.int32),
        pltpu.VMEM((2 * _G, 8 * _DIM), jnp.float32),
        pltpu.VMEM((2 * _G, 8 * _DIM), jnp.float32),
        pltpu.VMEM((_NREL // 8, 8 * _DIM), jnp.float32),
        pltpu.VMEM((_G, _DIM), jnp.float32),
        pltpu.SemaphoreType.DMA,
        pltpu.SemaphoreType.DMA,
    ],
)(_combine_body)


@jax.jit
def kernel(h, r, t, entity_embeddings, relation_embeddings):
    hi = h.astype(jnp.int32)
    ri = r.astype(jnp.int32)
    ti = t.astype(jnp.int32)
    ent3 = entity_embeddings.reshape(-1, 8, _DIM)
    gathered = _gather(hi.reshape(_NW, _BPW), ti.reshape(_NW, _BPW), ent3)
    g2 = gathered.reshape(-1, 8 * _DIM)
    rel2 = relation_embeddings.reshape(-1, 8 * _DIM)
    return _combine(hi, ri, ti, g2, rel2)


# speedup vs baseline: 6.9404x; 6.9404x over previous
import functools

import jax
import jax.numpy as jnp
from jax import lax
from jax.experimental import pallas as pl
from jax.experimental.pallas import tpu as pltpu
from jax.experimental.pallas import tpu_sc as plsc

_BATCH = 16384
_DIM = 64
_LANES = 16
_NUM_CORES = 2
_NUM_SUBCORES = 16
_NW = _NUM_CORES * _NUM_SUBCORES
_BPW = _BATCH // _NW
_CH = 128
_NCHUNK = _BPW // _CH


def _sc_body(h_hbm, r_hbm, t_hbm, ent_hbm, rel_hbm, out_hbm,
             hidx, ridx, tidx, hbuf, rbuf, tbuf, sem_in, sem_out):
    wid = lax.axis_index("s") * _NUM_CORES + lax.axis_index("c")
    base = wid * _BPW

    for j in range(_NCHUNK):
        off = base + j * _CH
        pltpu.sync_copy(h_hbm.at[pl.ds(off, _CH)], hidx.at[j])
        pltpu.sync_copy(r_hbm.at[pl.ds(off, _CH)], ridx.at[j])
        pltpu.sync_copy(t_hbm.at[pl.ds(off, _CH)], tidx.at[j])

    def fire(j):
        s = j % 2
        copies = (
            pltpu.async_copy(ent_hbm.at[hidx.at[j]], hbuf.at[s], sem_in),
            pltpu.async_copy(ent_hbm.at[tidx.at[j]], tbuf.at[s], sem_in),
            pltpu.async_copy(rel_hbm.at[ridx.at[j]], rbuf.at[s], sem_in),
        )
        return copies

    in_flight = [fire(0)]
    out_flight = [None, None]
    for j in range(_NCHUNK):
        s = j % 2
        if j + 1 < _NCHUNK:
            if out_flight[(j + 1) % 2] is not None:
                out_flight[(j + 1) % 2].wait()
                out_flight[(j + 1) % 2] = None
            in_flight.append(fire(j + 1))
        for c in in_flight.pop(0):
            c.wait()

        def row(i, _):
            for u in range(2):
                for c in range(_DIM // _LANES):
                    sl = pl.ds(c * _LANES, _LANES)
                    hbuf[s, 2 * i + u, sl] = (
                        hbuf[s, 2 * i + u, sl]
                        + rbuf[s, 2 * i + u, sl]
                        - tbuf[s, 2 * i + u, sl]
                    )
            return 0

        lax.fori_loop(0, _CH // 2, row, 0)
        out_flight[s] = pltpu.async_copy(
            hbuf.at[s], out_hbm.at[pl.ds(base + j * _CH, _CH)], sem_out)

    for s in range(2):
        if out_flight[s] is not None:
            out_flight[s].wait()


_trans_e = functools.partial(
    pl.kernel,
    mesh=plsc.VectorSubcoreMesh(core_axis_name="c", subcore_axis_name="s"),
    out_type=jax.ShapeDtypeStruct((_BATCH, _DIM), jnp.float32),
    scratch_types=[
        pltpu.VMEM((_NCHUNK, _CH), jnp.int32),
        pltpu.VMEM((_NCHUNK, _CH), jnp.int32),
        pltpu.VMEM((_NCHUNK, _CH), jnp.int32),
        pltpu.VMEM((2, _CH, _DIM), jnp.float32),
        pltpu.VMEM((2, _CH, _DIM), jnp.float32),
        pltpu.VMEM((2, _CH, _DIM), jnp.float32),
        pltpu.SemaphoreType.DMA,
        pltpu.SemaphoreType.DMA,
    ],
    compiler_params=pltpu.CompilerParams(use_tc_tiling_on_sc=False),
)(_sc_body)


@jax.jit
def kernel(h, r, t, entity_embeddings, relation_embeddings):
    return _trans_e(
        h.astype(jnp.int32),
        r.astype(jnp.int32),
        t.astype(jnp.int32),
        entity_embeddings,
        relation_embeddings,
    )
